# baseline (speedup 1.0000x reference)
import functools

import jax
import jax.numpy as jnp
from jax import lax
from jax.experimental import pallas as pl
from jax.experimental.pallas import tpu as pltpu
from jax.experimental.pallas import tpu_sc as plsc

MAX_V = 3
D = 128
B = 16384
NC, NS, L = 2, 16, 16
NW = NC * NS
BPW = B // NW
CB = 128
NCH = BPW // CB

_mesh = plsc.VectorSubcoreMesh(
    core_axis_name="c", subcore_axis_name="s", num_cores=NC, num_subcores=NS)


@functools.partial(
    pl.kernel,
    out_type=jax.ShapeDtypeStruct((B, D), jnp.float32),
    mesh=_mesh,
    scratch_types=[
        pltpu.VMEM((MAX_V, D), jnp.float32),
        pltpu.VMEM((L,), jnp.int32),
        pltpu.VMEM((CB, D), jnp.float32),
        pltpu.VMEM((CB, D), jnp.float32),
        pltpu.SemaphoreType.DMA,
        pltpu.SemaphoreType.DMA,
        pltpu.SemaphoreType.DMA,
        pltpu.SemaphoreType.DMA,
    ],
)
def _view_embed_kernel(gf_hbm, emb_hbm, idx_hbm, out_hbm,
                       emb_v, idx_v, buf0, buf1, si0, si1, so0, so1):
    wid = lax.axis_index("s") * NC + lax.axis_index("c")
    base = wid * BPW

    pltpu.sync_copy(emb_hbm, emb_v)
    pltpu.sync_copy(idx_hbm, idx_v)
    iv = idx_v[...]
    ev = []
    for j in range(D // L):
        r0 = emb_v[0, pl.ds(j * L, L)]
        r1 = emb_v[1, pl.ds(j * L, L)]
        r2 = emb_v[2, pl.ds(j * L, L)]
        ev.append(jnp.where(iv == 0, r0, jnp.where(iv == 1, r1, r2)))

    bufs = (buf0, buf1)
    isems = (si0, si1)
    osems = (so0, so1)

    def start_in(c):
        return pltpu.async_copy(
            gf_hbm.at[pl.ds(base + c * CB, CB)], bufs[c % 2], isems[c % 2])

    def start_out(c):
        return pltpu.async_copy(
            bufs[c % 2], out_hbm.at[pl.ds(base + c * CB, CB)], osems[c % 2])

    def compute(buf):
        def body(i, carry):
            for j in range(D // L):
                sl = (i, pl.ds(j * L, L))
                buf[sl] = buf[sl] + ev[j]
            return carry
        lax.fori_loop(0, CB, body, 0)

    in_h = [None] * NCH
    out_h = [None] * NCH
    in_h[0] = start_in(0)
    for c in range(NCH):
        if c + 1 < NCH:
            if c - 1 >= 0:
                out_h[c - 1].wait()
            in_h[c + 1] = start_in(c + 1)
        in_h[c].wait()
        compute(bufs[c % 2])
        out_h[c] = start_out(c)
    out_h[NCH - 2].wait()
    out_h[NCH - 1].wait()


def kernel(global_feat, embeddings, view_idx):
    idx = jnp.full((L,), view_idx, dtype=jnp.int32)
    return _view_embed_kernel(global_feat, embeddings, idx)
